# baseline (speedup 1.0000x reference)
import functools

import jax
import jax.numpy as jnp
import numpy as np
from jax import lax
from jax.experimental import pallas as pl
from jax.experimental.pallas import tpu as pltpu
from jax.experimental.pallas import tpu_sc as plsc

VOCAB = 1000000
D_MODEL = 128
B = 4
L = 2048
SCALE = float(np.sqrt(np.float64(D_MODEL)))
LN_1E4 = float(np.log(np.float64(10000.0)))

NUM_WORKERS = 32
POS_PER_W = L // NUM_WORKERS
HALF_POS = POS_PER_W // 2
LANES = 16
HALF = D_MODEL // 2
NCH = HALF // LANES
WID_BITS = 5


def _cmul(s1, c1, s2, c2):
    return s1 * c2 + c1 * s2, c1 * c2 - s1 * s2


def _sc_body(x_hbm, table_hbm, out_hbm, idx_v, g_v,
             sem_ix, sem_g0, sem_g1, sem_st):
    wid = lax.axis_index("s") * 2 + lax.axis_index("c")
    l0 = wid * POS_PER_W

    hi = [pltpu.async_copy(x_hbm.at[bb, pl.ds(l0, POS_PER_W)],
                           idx_v.at[pl.ds(bb * POS_PER_W, POS_PER_W)],
                           sem_ix)
          for bb in range(B)]
    h0, h1 = [], []
    for bb in range(B):
        hi[bb].wait()
        h0.append(pltpu.async_copy(
            table_hbm.at[idx_v.at[pl.ds(bb * POS_PER_W, HALF_POS)]],
            g_v.at[bb, pl.ds(0, HALF_POS)], sem_g0))
        h1.append(pltpu.async_copy(
            table_hbm.at[idx_v.at[pl.ds(bb * POS_PER_W + HALF_POS, HALF_POS)]],
            g_v.at[bb, pl.ds(HALF_POS, HALF_POS)], sem_g1))

    sr, cr = [], []
    for j in range(NCH):
        d = lax.iota(jnp.int32, LANES).astype(jnp.float32) + float(j * LANES)
        r = jnp.exp(d * (-LN_1E4 / HALF))
        x2 = r * r
        sp = 1.0 + x2 * (-1.0 / 6.0 + x2 * (1.0 / 120.0 + x2 * (
            -1.0 / 5040.0 + x2 * (1.0 / 362880.0))))
        cp = 1.0 + x2 * (-0.5 + x2 * (1.0 / 24.0 + x2 * (
            -1.0 / 720.0 + x2 * (1.0 / 40320.0 + x2 * (-1.0 / 3628800.0)))))
        sr.append(r * sp)
        cr.append(cp)

    p_s, p_c = list(sr), list(cr)
    for _ in range(6):
        for j in range(NCH):
            p_s[j], p_c[j] = _cmul(p_s[j], p_c[j], p_s[j], p_c[j])
    s = [jnp.zeros((LANES,), jnp.float32) for _ in range(NCH)]
    c = [jnp.ones((LANES,), jnp.float32) for _ in range(NCH)]
    for k in range(WID_BITS):
        bit = ((wid >> k) & 1).astype(jnp.float32)
        m = jnp.full((LANES,), 1.0, jnp.float32) * bit
        for j in range(NCH):
            ns, nc = _cmul(s[j], c[j], p_s[j], p_c[j])
            s[j] = s[j] + m * (ns - s[j])
            c[j] = c[j] + m * (nc - c[j])
        if k + 1 < WID_BITS:
            for j in range(NCH):
                p_s[j], p_c[j] = _cmul(p_s[j], p_c[j], p_s[j], p_c[j])

    def body(p, carry):
        sc = list(carry)
        for bb in range(B):
            for j in range(NCH):
                sl = pl.ds(j * LANES, LANES)
                g_v[bb, p, sl] = g_v[bb, p, sl] * SCALE + sc[j]
                slh = pl.ds(HALF + j * LANES, LANES)
                g_v[bb, p, slh] = g_v[bb, p, slh] * SCALE + sc[NCH + j]
        out = []
        for j in range(NCH):
            out.append(sc[j] * cr[j] + sc[NCH + j] * sr[j])
        for j in range(NCH):
            out.append(sc[NCH + j] * cr[j] - sc[j] * sr[j])
        return tuple(out)

    carry = tuple(s + c)
    QP = HALF_POS // 2
    stores = []
    for q in range(POS_PER_W // QP):
        if q == 0:
            for h in h0:
                h.wait()
        if q * QP == HALF_POS:
            for h in h1:
                h.wait()
        carry = lax.fori_loop(q * QP, (q + 1) * QP, body, carry)
        stores += [pltpu.async_copy(
            g_v.at[bb, pl.ds(q * QP, QP)],
            out_hbm.at[bb, pl.ds(l0 + q * QP, QP)], sem_st)
            for bb in range(B)]
    for h in stores:
        h.wait()


def kernel(x, table):
    sc_call = functools.partial(
        pl.kernel,
        out_type=jax.ShapeDtypeStruct((B, L, D_MODEL), jnp.float32),
        mesh=plsc.VectorSubcoreMesh(core_axis_name="c", subcore_axis_name="s"),
        scratch_types=[
            pltpu.VMEM((B * POS_PER_W,), jnp.int32),
            pltpu.VMEM((B, POS_PER_W, D_MODEL), jnp.float32),
            pltpu.SemaphoreType.DMA,
            pltpu.SemaphoreType.DMA,
            pltpu.SemaphoreType.DMA,
            pltpu.SemaphoreType.DMA,
        ],
    )(_sc_body)

    return sc_call(x, table)
